# Initial kernel scaffold; baseline (speedup 1.0000x reference)
import jax
import jax.numpy as jnp
from jax.experimental import pallas as pl


def kernel(input_features, neighbor_inds, guidance, weightnet):
    raise NotImplementedError("write your pallas kernel here")


# speedup vs baseline: 2.8139x; 2.8139x over previous
import functools

import jax
import jax.numpy as jnp
from jax import lax
from jax.experimental import pallas as pl
from jax.experimental.pallas import tpu as pltpu
from jax.experimental.pallas import tpu_sc as plsc

N_CORES = 2
N_SUBCORES = 16
NW = N_CORES * N_SUBCORES
CHUNK = 128


def _sc_gather(table, idx2d, total_rows):
    n, c = table.shape
    nch = idx2d.shape[0]
    iters = (nch + NW - 1) // NW
    mesh = plsc.VectorSubcoreMesh(core_axis_name="c", subcore_axis_name="s")

    @functools.partial(
        pl.kernel,
        mesh=mesh,
        out_type=jax.ShapeDtypeStruct((total_rows, c), jnp.float32),
        scratch_types=[
            pltpu.VMEM((CHUNK,), jnp.int32),
            pltpu.VMEM((CHUNK, c), jnp.float32),
            pltpu.SemaphoreType.DMA,
        ],
    )
    def k(table_hbm, idx_hbm, out_hbm, idx_v, rows_v, sem):
        wid = lax.axis_index("s") * N_CORES + lax.axis_index("c")

        def step(i, carry):
            ch = wid + i * NW

            @pl.when(ch < nch)
            def _():
                pltpu.sync_copy(idx_hbm.at[ch], idx_v)
                pltpu.async_copy(table_hbm.at[idx_v], rows_v, sem).wait()
                pltpu.sync_copy(rows_v, out_hbm.at[pl.ds(ch * CHUNK, CHUNK)])

            return carry

        lax.fori_loop(0, iters, step, 0)

    return k(table, idx2d)


def _tc_combine(gathered, guid2, w_t, m, kk, c, cmid, bm):
    def body(g_ref, guid_ref, wt_ref, out_ref):
        row = lax.broadcasted_iota(jnp.int32, (8, c), 0)
        col = lax.broadcasted_iota(jnp.int32, (8, c), 1)
        exp = (col // (c // 8) == row).astype(jnp.float32)
        gexp = jnp.dot(guid_ref[...], exp, preferred_element_type=jnp.float32)
        g3 = (g_ref[...] * gexp).reshape(bm, kk, c)
        wt = wt_ref[...]
        acc = jnp.zeros((bm, cmid, c), jnp.float32)
        for k in range(kk):
            a = jnp.broadcast_to(g3[:, k : k + 1, :], (bm, cmid, c))
            b = jnp.broadcast_to(wt[:, :, k : k + 1], (bm, cmid, c))
            acc = acc + a * b
        out_ref[...] = acc

    grid = (m // bm,)
    return pl.pallas_call(
        body,
        grid=grid,
        in_specs=[
            pl.BlockSpec((bm * kk, c), lambda i: (i, 0)),
            pl.BlockSpec((bm * kk, 8), lambda i: (i, 0)),
            pl.BlockSpec((bm, cmid, kk), lambda i: (i, 0, 0)),
        ],
        out_specs=pl.BlockSpec((bm, cmid, c), lambda i: (i, 0, 0)),
        out_shape=jax.ShapeDtypeStruct((m, cmid, c), jnp.float32),
    )(gathered, guid2, w_t)


def kernel(input_features, neighbor_inds, guidance, weightnet):
    b, n, c = input_features.shape
    _, m, kk = neighbor_inds.shape
    h = guidance.shape[-1]
    cmid = weightnet.shape[-1]
    assert b == 1

    table = input_features[0]
    idx2d = neighbor_inds.reshape(-1, CHUNK)
    gathered = _sc_gather(table, idx2d, m * kk)

    guid2 = guidance.reshape(m * kk, h)
    w_t = jnp.swapaxes(weightnet[0], 1, 2)

    out_t = _tc_combine(gathered, guid2, w_t, m, kk, c, cmid, bm=80)
    return jnp.swapaxes(out_t, 1, 2).reshape(b, m, c * cmid)


# speedup vs baseline: 4.9304x; 1.7521x over previous
import functools

import jax
import jax.numpy as jnp
from jax import lax
from jax.experimental import pallas as pl
from jax.experimental.pallas import tpu as pltpu
from jax.experimental.pallas import tpu_sc as plsc

N_CORES = 2
N_SUBCORES = 16
NW = N_CORES * N_SUBCORES
CHUNK = 128


def _sc_gather(table, idx2d, total_rows):
    n, c = table.shape
    nch = idx2d.shape[0]
    iters = (nch + NW - 1) // NW
    mesh = plsc.VectorSubcoreMesh(core_axis_name="c", subcore_axis_name="s")

    @functools.partial(
        pl.kernel,
        mesh=mesh,
        out_type=jax.ShapeDtypeStruct((total_rows, c), jnp.float32),
        scratch_types=[
            pltpu.VMEM((CHUNK,), jnp.int32),
            pltpu.VMEM((CHUNK, c), jnp.float32),
            pltpu.SemaphoreType.DMA,
        ],
    )
    def k(table_hbm, idx_hbm, out_hbm, idx_v, rows_v, sem):
        wid = lax.axis_index("s") * N_CORES + lax.axis_index("c")

        def step(i, carry):
            ch = wid + i * NW

            @pl.when(ch < nch)
            def _():
                pltpu.sync_copy(idx_hbm.at[ch], idx_v)
                pltpu.async_copy(table_hbm.at[idx_v], rows_v, sem).wait()
                pltpu.sync_copy(rows_v, out_hbm.at[pl.ds(ch * CHUNK, CHUNK)])

            return carry

        lax.fori_loop(0, iters, step, 0)

    return k(table, idx2d)


def _tc_combine(gathered, guid2, w2d, m, kk, c, cmid, bm, grp):
    ng = bm // grp
    rg = grp * kk
    dg = grp * cmid

    def body(g_ref, guid_ref, w_ref, out_ref):
        row8 = lax.broadcasted_iota(jnp.int32, (8, c), 0)
        col8 = lax.broadcasted_iota(jnp.int32, (8, c), 1)
        exp = (col8 // (c // 8) == row8).astype(jnp.float32)
        gexp = jnp.dot(guid_ref[...], exp, preferred_element_type=jnp.float32)
        g3 = g_ref[...] * gexp
        t2r = lax.broadcasted_iota(jnp.int32, (cmid, dg), 0)
        t2c = lax.broadcasted_iota(jnp.int32, (cmid, dg), 1)
        t2 = (t2r == t2c % cmid).astype(jnp.float32)
        mkr = lax.broadcasted_iota(jnp.int32, (rg, dg), 0)
        mkc = lax.broadcasted_iota(jnp.int32, (rg, dg), 1)
        mask_t = (mkr // kk == mkc // cmid).astype(jnp.float32)
        for gg in range(ng):
            g3g = g3[gg * rg : (gg + 1) * rg, :]
            w3g = w_ref[gg * rg : (gg + 1) * rg, :]
            w2t = jnp.dot(w3g, t2, preferred_element_type=jnp.float32) * mask_t
            out_g = lax.dot_general(
                w2t, g3g, (((0,), (0,)), ((), ())),
                preferred_element_type=jnp.float32)
            out_ref[gg * dg : (gg + 1) * dg, :] = out_g

    grid = (m // bm,)
    return pl.pallas_call(
        body,
        grid=grid,
        in_specs=[
            pl.BlockSpec((bm * kk, c), lambda i: (i, 0)),
            pl.BlockSpec((bm * kk, 8), lambda i: (i, 0)),
            pl.BlockSpec((bm * kk, cmid), lambda i: (i, 0)),
        ],
        out_specs=pl.BlockSpec((bm * cmid, c), lambda i: (i, 0)),
        out_shape=jax.ShapeDtypeStruct((m * cmid, c), jnp.float32),
    )(gathered, guid2, w2d)


def kernel(input_features, neighbor_inds, guidance, weightnet):
    b, n, c = input_features.shape
    _, m, kk = neighbor_inds.shape
    h = guidance.shape[-1]
    cmid = weightnet.shape[-1]
    assert b == 1

    table = input_features[0]
    idx2d = neighbor_inds.reshape(-1, CHUNK)
    gathered = _sc_gather(table, idx2d, m * kk)

    guid2 = guidance.reshape(m * kk, h)
    w2d = weightnet.reshape(m * kk, cmid)

    out_t2 = _tc_combine(gathered, guid2, w2d, m, kk, c, cmid, bm=80, grp=8)
    return jnp.swapaxes(out_t2.reshape(m, cmid, c), 1, 2).reshape(b, m, c * cmid)
